# Initial kernel scaffold; baseline (speedup 1.0000x reference)
import jax
import jax.numpy as jnp
from jax.experimental import pallas as pl


def kernel(x, ve, x0, lambdas, Wq, Wk, Wv, Wo, k_pe, v_pe, Wkc, Wvc, Wg, Wfc, Wproj, sliding_window_flex_mask, fine_selection_flex_mask):
    raise NotImplementedError("write your pallas kernel here")


# speedup vs baseline: 1.2202x; 1.2202x over previous
import jax
import jax.numpy as jnp
from jax.experimental import pallas as pl

S = 2048
DIM = 768
H = 12
DH = 64
BLK = 4
NB = S // BLK
WIN = 32
QC = 256
NQ = S // QC
GCOL = 128
SCALE = DH ** -0.5


def _prep_kernel(x_ref, x0_ref, lam_ref, w_ref, x1_ref, y_ref):
    lam0 = lam_ref[0, 0]
    lam1 = lam_ref[0, 1]
    x1 = lam0 * x_ref[...] + lam1 * x0_ref[...]
    x1_ref[...] = x1
    h = x1 * jax.lax.rsqrt(jnp.mean(x1 * x1, axis=-1, keepdims=True) + 1e-6)
    y = jnp.dot(h, w_ref[...], preferred_element_type=jnp.float32)
    y_ref[:, : 3 * DIM] = y[:, : 3 * DIM]
    y_ref[:, 3 * DIM :] = jax.nn.sigmoid(y[:, 3 * DIM :])


def _ckv_kernel(kh_ref, vh_ref, wkc_ref, wvc_ref, kpe_ref, vpe_ref, ck_ref, cv_ref):
    pe_k = jnp.dot(kpe_ref[...], wkc_ref[...], preferred_element_type=jnp.float32)
    pe_v = jnp.dot(vpe_ref[...], wvc_ref[...], preferred_element_type=jnp.float32)
    ck_ref[0] = jnp.dot(kh_ref[0], wkc_ref[...], preferred_element_type=jnp.float32) + pe_k
    cv_ref[0] = jnp.dot(vh_ref[0], wvc_ref[...], preferred_element_type=jnp.float32) + pe_v


def _attn_kernel(q_ref, k_ref, v_ref, ck_ref, cv_ref, fm_ref, g_ref, out_ref):
    qc = pl.program_id(1)
    q = q_ref[0]
    k = k_ref[0]
    v = v_ref[0]
    sim = jax.lax.dot_general(q, k, (((1,), (1,)), ((), ()))) * SCALE
    row = qc * QC + jax.lax.broadcasted_iota(jnp.int32, (QC, S), 0)
    col = jax.lax.broadcasted_iota(jnp.int32, (QC, S), 1)
    causal = col <= row

    fm = fm_ref[...].astype(jnp.float32)
    sf = sim * fm + (fm - 1.0) * 1e9
    mf = jnp.max(sf, axis=-1, keepdims=True)
    pf = jnp.exp(sf - mf)
    f_out = jax.lax.dot_general(pf, v, (((1,), (0,)), ((), ()))) / jnp.sum(
        pf, axis=-1, keepdims=True
    )

    sm = causal & ((row - col) < WIN)
    ss = jnp.where(sm, sim, -1e9)
    ms = jnp.max(ss, axis=-1, keepdims=True)
    ps = jnp.exp(ss - ms)
    s_out = jax.lax.dot_general(ps, v, (((1,), (0,)), ((), ()))) / jnp.sum(
        ps, axis=-1, keepdims=True
    )

    ck = ck_ref[0]
    cv = cv_ref[0]
    simc = jax.lax.dot_general(q, ck, (((1,), (1,)), ((), ()))) * SCALE
    rowc = qc * QC + jax.lax.broadcasted_iota(jnp.int32, (QC, NB), 0)
    colc = jax.lax.broadcasted_iota(jnp.int32, (QC, NB), 1)
    cmask = ((colc + 1) * BLK - 1) <= rowc
    sc = jnp.where(cmask, simc, -1e9)
    mc = jnp.maximum(jnp.max(sc, axis=-1, keepdims=True), 0.0)
    pc = jnp.exp(sc - mc)
    den = jnp.sum(pc, axis=-1, keepdims=True) + jnp.exp(-mc)
    c_out = jax.lax.dot_general(pc, cv, (((1,), (0,)), ((), ()))) / den

    g = g_ref[0]
    out_ref[0] = g[:, 0:1] * c_out + g[:, 1:2] * f_out + g[:, 2:3] * s_out


def _mlp_kernel(attn_ref, x1_ref, wo_ref, wfc_ref, wproj_ref, y_ref):
    x2 = x1_ref[...] + jnp.dot(attn_ref[...], wo_ref[...], preferred_element_type=jnp.float32)
    h2 = x2 * jax.lax.rsqrt(jnp.mean(x2 * x2, axis=-1, keepdims=True) + 1e-6)
    u = jnp.dot(h2, wfc_ref[...], preferred_element_type=jnp.float32)
    u = jnp.square(jnp.maximum(u, 0.0))
    y_ref[...] = x2 + jnp.dot(u, wproj_ref[...], preferred_element_type=jnp.float32)


def kernel(x, ve, x0, lambdas, Wq, Wk, Wv, Wo, k_pe, v_pe, Wkc, Wvc, Wg, Wfc, Wproj,
           sliding_window_flex_mask, fine_selection_flex_mask):
    del ve, sliding_window_flex_mask
    x2d = x[0]
    x02d = x0[0]
    w_all = jnp.concatenate(
        [Wq, Wk, Wv, jnp.pad(Wg, ((0, 0), (0, GCOL - 3 * H)))], axis=1
    )
    lam2 = lambdas.reshape(1, 2)

    x1, y = pl.pallas_call(
        _prep_kernel,
        grid=(NQ,),
        in_specs=[
            pl.BlockSpec((QC, DIM), lambda i: (i, 0)),
            pl.BlockSpec((QC, DIM), lambda i: (i, 0)),
            pl.BlockSpec((1, 2), lambda i: (0, 0)),
            pl.BlockSpec((DIM, 3 * DIM + GCOL), lambda i: (0, 0)),
        ],
        out_specs=[
            pl.BlockSpec((QC, DIM), lambda i: (i, 0)),
            pl.BlockSpec((QC, 3 * DIM + GCOL), lambda i: (i, 0)),
        ],
        out_shape=[
            jax.ShapeDtypeStruct((S, DIM), jnp.float32),
            jax.ShapeDtypeStruct((S, 3 * DIM + GCOL), jnp.float32),
        ],
    )(x2d, x02d, lam2, w_all)

    qkv = y[:, : 3 * DIM].reshape(S, 3, H, DH).transpose(1, 2, 0, 3)
    q_hm = qkv[0]
    k_hm = qkv[1]
    v_hm = qkv[2]
    g_hm = y[:, 3 * DIM : 3 * DIM + 3 * H].reshape(S, H, 3).transpose(1, 0, 2)
    kh = k_hm.reshape(H, NB, BLK * DH)
    vh = v_hm.reshape(H, NB, BLK * DH)

    ck, cv = pl.pallas_call(
        _ckv_kernel,
        grid=(H,),
        in_specs=[
            pl.BlockSpec((1, NB, BLK * DH), lambda h: (h, 0, 0)),
            pl.BlockSpec((1, NB, BLK * DH), lambda h: (h, 0, 0)),
            pl.BlockSpec((BLK * DH, DH), lambda h: (0, 0)),
            pl.BlockSpec((BLK * DH, DH), lambda h: (0, 0)),
            pl.BlockSpec((1, BLK * DH), lambda h: (0, 0)),
            pl.BlockSpec((1, BLK * DH), lambda h: (0, 0)),
        ],
        out_specs=[
            pl.BlockSpec((1, NB, DH), lambda h: (h, 0, 0)),
            pl.BlockSpec((1, NB, DH), lambda h: (h, 0, 0)),
        ],
        out_shape=[
            jax.ShapeDtypeStruct((H, NB, DH), jnp.float32),
            jax.ShapeDtypeStruct((H, NB, DH), jnp.float32),
        ],
    )(kh, vh, Wkc, Wvc, k_pe.reshape(1, BLK * DH), v_pe.reshape(1, BLK * DH))

    fm8 = fine_selection_flex_mask.astype(jnp.int8)

    attn = pl.pallas_call(
        _attn_kernel,
        grid=(H, NQ),
        in_specs=[
            pl.BlockSpec((1, QC, DH), lambda h, i: (h, i, 0)),
            pl.BlockSpec((1, S, DH), lambda h, i: (h, 0, 0)),
            pl.BlockSpec((1, S, DH), lambda h, i: (h, 0, 0)),
            pl.BlockSpec((1, NB, DH), lambda h, i: (h, 0, 0)),
            pl.BlockSpec((1, NB, DH), lambda h, i: (h, 0, 0)),
            pl.BlockSpec((QC, S), lambda h, i: (i, 0)),
            pl.BlockSpec((1, QC, 3), lambda h, i: (h, i, 0)),
        ],
        out_specs=pl.BlockSpec((1, QC, DH), lambda h, i: (h, i, 0)),
        out_shape=jax.ShapeDtypeStruct((H, S, DH), jnp.float32),
    )(q_hm, k_hm, v_hm, ck, cv, fm8, g_hm)

    attn2d = attn.transpose(1, 0, 2).reshape(S, H * DH)

    out = pl.pallas_call(
        _mlp_kernel,
        grid=(NQ,),
        in_specs=[
            pl.BlockSpec((QC, H * DH), lambda i: (i, 0)),
            pl.BlockSpec((QC, DIM), lambda i: (i, 0)),
            pl.BlockSpec((H * DH, DIM), lambda i: (0, 0)),
            pl.BlockSpec((DIM, 4 * DIM), lambda i: (0, 0)),
            pl.BlockSpec((4 * DIM, DIM), lambda i: (0, 0)),
        ],
        out_specs=pl.BlockSpec((QC, DIM), lambda i: (i, 0)),
        out_shape=jax.ShapeDtypeStruct((S, DIM), jnp.float32),
    )(attn2d, x1, Wo, Wfc, Wproj)

    return out[None]


# speedup vs baseline: 1.2797x; 1.0488x over previous
import jax
import jax.numpy as jnp
from jax.experimental import pallas as pl

S = 2048
DIM = 768
H = 12
DH = 64
BLK = 4
NB = S // BLK
WIN = 32
QC = 256
NQ = S // QC
GCOL = 128
SCALE = DH ** -0.5
F32 = jnp.float32
BF16 = jnp.bfloat16


def _prep_kernel(x_ref, x0_ref, lam_ref, w_ref, x1_ref, y_ref):
    lam0 = lam_ref[0, 0]
    lam1 = lam_ref[0, 1]
    x1 = lam0 * x_ref[...] + lam1 * x0_ref[...]
    x1_ref[...] = x1
    h = x1 * jax.lax.rsqrt(jnp.mean(x1 * x1, axis=-1, keepdims=True) + 1e-6)
    y = jnp.dot(h.astype(BF16), w_ref[...], preferred_element_type=F32)
    y_ref[:, : 3 * DIM] = y[:, : 3 * DIM]
    y_ref[:, 3 * DIM :] = jax.nn.sigmoid(y[:, 3 * DIM :])


def _ckv_kernel(kh_ref, vh_ref, wkc_ref, wvc_ref, kpe_ref, vpe_ref, ck_ref, cv_ref):
    pe_k = jnp.dot(kpe_ref[...], wkc_ref[...], preferred_element_type=F32)
    pe_v = jnp.dot(vpe_ref[...], wvc_ref[...], preferred_element_type=F32)
    ck_ref[0] = (jnp.dot(kh_ref[0], wkc_ref[...], preferred_element_type=F32) + pe_k).astype(BF16)
    cv_ref[0] = (jnp.dot(vh_ref[0], wvc_ref[...], preferred_element_type=F32) + pe_v).astype(BF16)


def _attn_kernel(q_ref, k_ref, v_ref, ck_ref, cv_ref, fm_ref, g_ref, out_ref):
    qc = pl.program_id(1)
    q = q_ref[0]
    k = k_ref[0]
    v = v_ref[0]
    sim = jax.lax.dot_general(q, k, (((1,), (1,)), ((), ())),
                              preferred_element_type=F32) * SCALE
    row = qc * QC + jax.lax.broadcasted_iota(jnp.int32, (QC, S), 0)
    col = jax.lax.broadcasted_iota(jnp.int32, (QC, S), 1)
    causal = col <= row

    fm = fm_ref[...].astype(F32)
    sf = sim * fm + (fm - 1.0) * 1e9
    mf = jnp.max(sf, axis=-1, keepdims=True)
    pf = jnp.exp(sf - mf)
    f_out = jax.lax.dot_general(pf.astype(BF16), v, (((1,), (0,)), ((), ())),
                                preferred_element_type=F32) / jnp.sum(
        pf, axis=-1, keepdims=True
    )

    sm = causal & ((row - col) < WIN)
    ss = jnp.where(sm, sim, -1e9)
    ms = jnp.max(ss, axis=-1, keepdims=True)
    ps = jnp.exp(ss - ms)
    s_out = jax.lax.dot_general(ps.astype(BF16), v, (((1,), (0,)), ((), ())),
                                preferred_element_type=F32) / jnp.sum(
        ps, axis=-1, keepdims=True
    )

    ck = ck_ref[0]
    cv = cv_ref[0]
    simc = jax.lax.dot_general(q, ck, (((1,), (1,)), ((), ())),
                               preferred_element_type=F32) * SCALE
    rowc = qc * QC + jax.lax.broadcasted_iota(jnp.int32, (QC, NB), 0)
    colc = jax.lax.broadcasted_iota(jnp.int32, (QC, NB), 1)
    cmask = ((colc + 1) * BLK - 1) <= rowc
    sc = jnp.where(cmask, simc, -1e9)
    mc = jnp.maximum(jnp.max(sc, axis=-1, keepdims=True), 0.0)
    pc = jnp.exp(sc - mc)
    den = jnp.sum(pc, axis=-1, keepdims=True) + jnp.exp(-mc)
    c_out = jax.lax.dot_general(pc.astype(BF16), cv, (((1,), (0,)), ((), ())),
                                preferred_element_type=F32) / den

    g = g_ref[0]
    out_ref[0] = g[:, 0:1] * c_out + g[:, 1:2] * f_out + g[:, 2:3] * s_out


def _mlp_kernel(attn_ref, x1_ref, wo_ref, wfc_ref, wproj_ref, y_ref):
    x2 = x1_ref[...] + jnp.dot(attn_ref[...], wo_ref[...], preferred_element_type=F32)
    h2 = x2 * jax.lax.rsqrt(jnp.mean(x2 * x2, axis=-1, keepdims=True) + 1e-6)
    u = jnp.dot(h2.astype(BF16), wfc_ref[...], preferred_element_type=F32)
    u = jnp.square(jnp.maximum(u, 0.0))
    y_ref[...] = x2 + jnp.dot(u.astype(BF16), wproj_ref[...], preferred_element_type=F32)


def kernel(x, ve, x0, lambdas, Wq, Wk, Wv, Wo, k_pe, v_pe, Wkc, Wvc, Wg, Wfc, Wproj,
           sliding_window_flex_mask, fine_selection_flex_mask):
    del ve, sliding_window_flex_mask
    x2d = x[0]
    x02d = x0[0]
    w_all = jnp.concatenate(
        [Wq, Wk, Wv, jnp.pad(Wg, ((0, 0), (0, GCOL - 3 * H)))], axis=1
    ).astype(BF16)
    lam2 = lambdas.reshape(1, 2)

    x1, y = pl.pallas_call(
        _prep_kernel,
        grid=(NQ,),
        in_specs=[
            pl.BlockSpec((QC, DIM), lambda i: (i, 0)),
            pl.BlockSpec((QC, DIM), lambda i: (i, 0)),
            pl.BlockSpec((1, 2), lambda i: (0, 0)),
            pl.BlockSpec((DIM, 3 * DIM + GCOL), lambda i: (0, 0)),
        ],
        out_specs=[
            pl.BlockSpec((QC, DIM), lambda i: (i, 0)),
            pl.BlockSpec((QC, 3 * DIM + GCOL), lambda i: (i, 0)),
        ],
        out_shape=[
            jax.ShapeDtypeStruct((S, DIM), F32),
            jax.ShapeDtypeStruct((S, 3 * DIM + GCOL), F32),
        ],
    )(x2d, x02d, lam2, w_all)

    qkv = y[:, : 3 * DIM].reshape(S, 3, H, DH).transpose(1, 2, 0, 3)
    qkv16 = qkv.astype(BF16)
    q_hm = qkv16[0]
    k_hm = qkv16[1]
    v_hm = qkv16[2]
    g_hm = y[:, 3 * DIM : 3 * DIM + 3 * H].reshape(S, H, 3).transpose(1, 0, 2)
    kh = k_hm.reshape(H, NB, BLK * DH)
    vh = v_hm.reshape(H, NB, BLK * DH)

    ck, cv = pl.pallas_call(
        _ckv_kernel,
        grid=(H,),
        in_specs=[
            pl.BlockSpec((1, NB, BLK * DH), lambda h: (h, 0, 0)),
            pl.BlockSpec((1, NB, BLK * DH), lambda h: (h, 0, 0)),
            pl.BlockSpec((BLK * DH, DH), lambda h: (0, 0)),
            pl.BlockSpec((BLK * DH, DH), lambda h: (0, 0)),
            pl.BlockSpec((1, BLK * DH), lambda h: (0, 0)),
            pl.BlockSpec((1, BLK * DH), lambda h: (0, 0)),
        ],
        out_specs=[
            pl.BlockSpec((1, NB, DH), lambda h: (h, 0, 0)),
            pl.BlockSpec((1, NB, DH), lambda h: (h, 0, 0)),
        ],
        out_shape=[
            jax.ShapeDtypeStruct((H, NB, DH), BF16),
            jax.ShapeDtypeStruct((H, NB, DH), BF16),
        ],
    )(kh, vh, Wkc.astype(BF16), Wvc.astype(BF16),
      k_pe.reshape(1, BLK * DH).astype(BF16), v_pe.reshape(1, BLK * DH).astype(BF16))

    fm8 = fine_selection_flex_mask.astype(jnp.int8)

    attn = pl.pallas_call(
        _attn_kernel,
        grid=(H, NQ),
        in_specs=[
            pl.BlockSpec((1, QC, DH), lambda h, i: (h, i, 0)),
            pl.BlockSpec((1, S, DH), lambda h, i: (h, 0, 0)),
            pl.BlockSpec((1, S, DH), lambda h, i: (h, 0, 0)),
            pl.BlockSpec((1, NB, DH), lambda h, i: (h, 0, 0)),
            pl.BlockSpec((1, NB, DH), lambda h, i: (h, 0, 0)),
            pl.BlockSpec((QC, S), lambda h, i: (i, 0)),
            pl.BlockSpec((1, QC, 3), lambda h, i: (h, i, 0)),
        ],
        out_specs=pl.BlockSpec((1, QC, DH), lambda h, i: (h, i, 0)),
        out_shape=jax.ShapeDtypeStruct((H, S, DH), F32),
    )(q_hm, k_hm, v_hm, ck, cv, fm8, g_hm)

    attn2d = attn.transpose(1, 0, 2).reshape(S, H * DH).astype(BF16)

    out = pl.pallas_call(
        _mlp_kernel,
        grid=(NQ,),
        in_specs=[
            pl.BlockSpec((QC, H * DH), lambda i: (i, 0)),
            pl.BlockSpec((QC, DIM), lambda i: (i, 0)),
            pl.BlockSpec((H * DH, DIM), lambda i: (0, 0)),
            pl.BlockSpec((DIM, 4 * DIM), lambda i: (0, 0)),
            pl.BlockSpec((4 * DIM, DIM), lambda i: (0, 0)),
        ],
        out_specs=pl.BlockSpec((QC, DIM), lambda i: (i, 0)),
        out_shape=jax.ShapeDtypeStruct((S, DIM), F32),
    )(attn2d, x1, Wo.astype(BF16), Wfc.astype(BF16), Wproj.astype(BF16))

    return out[None]
